# baseline (speedup 1.0000x reference)
import functools

import jax
import jax.numpy as jnp
from jax import lax
from jax.experimental import pallas as pl
from jax.experimental.pallas import tpu as pltpu
from jax.experimental.pallas import tpu_sc as plsc

BATCH = 16384
EMBED_DIM = 32
NUM_CORES = 2
B_PER_C = BATCH // NUM_CORES


def kernel(indices, table):
    idx = indices.astype(jnp.int32)
    mesh = plsc.ScalarSubcoreMesh(axis_name="core", num_cores=NUM_CORES)

    @functools.partial(
        pl.kernel,
        mesh=mesh,
        out_type=jax.ShapeDtypeStruct((BATCH, EMBED_DIM), jnp.float32),
        scratch_types=[
            pltpu.SMEM((B_PER_C,), jnp.int32),
            pltpu.SemaphoreType.DMA,
            pltpu.SemaphoreType.DMA,
        ],
    )
    def gather_kernel(tab_hbm, idx_hbm, out_hbm, idx_s, sem_i, sem):
        cid = lax.axis_index("core")
        base = cid * B_PER_C
        pltpu.async_copy(idx_hbm.at[pl.ds(base, B_PER_C)], idx_s, sem_i).wait()

        @pl.loop(0, B_PER_C)
        def _fire(i):
            row = idx_s[i]
            pltpu.async_copy(
                tab_hbm.at[pl.ds(row, 1)],
                out_hbm.at[pl.ds(base + i, 1)],
                sem,
            )

        @pl.loop(0, B_PER_C)
        def _drain(i):
            pltpu.make_async_copy(
                tab_hbm.at[pl.ds(0, 1)],
                out_hbm.at[pl.ds(base, 1)],
                sem,
            ).wait()

    return gather_kernel(table, idx)
